# Initial kernel scaffold; baseline (speedup 1.0000x reference)
import jax
import jax.numpy as jnp
from jax.experimental import pallas as pl


def kernel(loc, new_loc, features, W1, gamma1, beta1):
    raise NotImplementedError("write your pallas kernel here")


# speedup vs baseline: 8.6603x; 8.6603x over previous
import functools

import jax
import jax.numpy as jnp
from jax import lax
from jax.experimental import pallas as pl
from jax.experimental.pallas import tpu as pltpu
from jax.experimental.pallas import tpu_sc as plsc

B, N, M, K, D, C = 4, 8192, 1024, 32, 3, 32
CIN = D + C
CPAD = 48
CO = 64
QT = 128
RT = QT * K

SC_CORES = 2
SC_SUBCORES = 16
NW = SC_CORES * SC_SUBCORES
ROWS = B * M * K
ROWS_PER_W = ROWS // NW
GCHUNK = 128
NCHUNK = ROWS_PER_W // GCHUNK


def _topk_body(locT_ref, q_ref, idx_ref, d_ref):
    b = pl.program_id(0)
    p = locT_ref[0]
    px, py, pz = p[0:1, :], p[1:2, :], p[2:3, :]
    q = q_ref[0]
    qx, qy, qz = q[:, 0:1], q[:, 1:2], q[:, 2:3]
    p2 = px * px + py * py + pz * pz
    q2 = qx * qx + qy * qy + qz * qz
    bf = lambda x: x.astype(jnp.bfloat16).astype(jnp.float32)
    cross = bf(qx) * bf(px) + bf(qy) * bf(py) + bf(qz) * bf(pz)
    d_ref[...] = jnp.maximum(q2 + p2 - 2.0 * cross, 0.0)

    iota_n = lax.broadcasted_iota(jnp.int32, (1, N), 1)
    iota_k = lax.broadcasted_iota(jnp.int32, (1, K), 1)
    big = jnp.int32(2**30)

    def body(i, acc):
        d = d_ref[...]
        m = jnp.min(d, axis=1, keepdims=True)
        cand = jnp.where(d == m, iota_n, big)
        nidx = jnp.min(cand, axis=1, keepdims=True)
        d_ref[...] = jnp.where(iota_n == nidx, jnp.float32(jnp.inf), d)
        return jnp.where(iota_k == i, nidx, acc)

    acc = lax.fori_loop(0, K, body, jnp.zeros((QT, K), jnp.int32))
    idx_ref[0] = acc + b * N


def _topk_call(locT, new_loc):
    return pl.pallas_call(
        _topk_body,
        grid=(B, M // QT),
        in_specs=[
            pl.BlockSpec((1, D, N), lambda b, j: (b, 0, 0)),
            pl.BlockSpec((1, QT, D), lambda b, j: (b, j, 0)),
        ],
        out_specs=pl.BlockSpec((1, QT, K), lambda b, j: (b, j, 0)),
        out_shape=jax.ShapeDtypeStruct((B, M, K), jnp.int32),
        scratch_shapes=[pltpu.VMEM((QT, N), jnp.float32)],
    )(locT, new_loc)


def _gather_kernel(idx_hbm, table_hbm, out_hbm, idx_v, rows_v, sem):
    wid = lax.axis_index("s") * SC_CORES + lax.axis_index("c")
    base = wid * ROWS_PER_W
    pltpu.sync_copy(idx_hbm.at[wid], idx_v)
    for cidx in range(NCHUNK):
        pltpu.async_copy(table_hbm.at[idx_v.at[cidx]], rows_v, sem).wait()
        pltpu.sync_copy(rows_v, out_hbm.at[pl.ds(base + cidx * GCHUNK, GCHUNK)])


def _gather_call(idx_flat, table):
    mesh = plsc.VectorSubcoreMesh(core_axis_name="c", subcore_axis_name="s")
    f = pl.kernel(
        _gather_kernel,
        out_type=jax.ShapeDtypeStruct((ROWS, CPAD), jnp.float32),
        mesh=mesh,
        compiler_params=pltpu.CompilerParams(use_tc_tiling_on_sc=False),
        scratch_types=[
            pltpu.VMEM((NCHUNK, GCHUNK), jnp.int32),
            pltpu.VMEM((GCHUNK, CPAD), jnp.float32),
            pltpu.SemaphoreType.DMA,
        ],
    )
    return f(idx_flat.reshape(NW, NCHUNK, GCHUNK), table)


def _mlp_body(g_ref, w_ref, q_ref, maxo_ref, mino_ref, stats_ref):
    t = pl.program_id(0)
    w = w_ref[...].astype(jnp.bfloat16)
    qpad = jnp.concatenate(
        [q_ref[...], jnp.zeros((QT, CPAD - D), jnp.float32)], axis=1)
    h = g_ref[...].reshape(QT, K, CPAD) - qpad[:, None, :]
    hb = h.reshape(RT, CPAD).astype(jnp.bfloat16)
    y = jnp.dot(hb, w, preferred_element_type=jnp.float32)
    y3 = y.reshape(QT, K, CO)
    maxo_ref[...] = jnp.max(y3, axis=1)
    mino_ref[...] = jnp.min(y3, axis=1)
    s = jnp.sum(y3, axis=(0, 1)).reshape(1, CO)
    s2 = jnp.sum(y3 * y3, axis=(0, 1)).reshape(1, CO)

    @pl.when(t == 0)
    def _():
        stats_ref[...] = jnp.zeros((8, CO), jnp.float32)

    stats_ref[0:1, :] += s
    stats_ref[1:2, :] += s2


def _mlp_call(g, w1p, q_flat):
    return pl.pallas_call(
        _mlp_body,
        grid=(ROWS // RT,),
        in_specs=[
            pl.BlockSpec((RT, CPAD), lambda t: (t, 0)),
            pl.BlockSpec((CPAD, CO), lambda t: (0, 0)),
            pl.BlockSpec((QT, D), lambda t: (t, 0)),
        ],
        out_specs=[
            pl.BlockSpec((QT, CO), lambda t: (t, 0)),
            pl.BlockSpec((QT, CO), lambda t: (t, 0)),
            pl.BlockSpec((8, CO), lambda t: (0, 0)),
        ],
        out_shape=[
            jax.ShapeDtypeStruct((B * M, CO), jnp.float32),
            jax.ShapeDtypeStruct((B * M, CO), jnp.float32),
            jax.ShapeDtypeStruct((8, CO), jnp.float32),
        ],
    )(g, w1p, q_flat)


def _fin_body(maxo_ref, mino_ref, stats_ref, g_ref, b_ref, out_ref):
    cnt = jnp.float32(ROWS)
    mean = stats_ref[0:1, :] / cnt
    var = stats_ref[1:2, :] / cnt - mean * mean
    scale = g_ref[...] * lax.rsqrt(var + 1e-5)
    y = jnp.where(scale > 0.0, maxo_ref[...], mino_ref[...])
    out_ref[...] = jnp.maximum((y - mean) * scale + b_ref[...], 0.0)


def _fin_call(maxo, mino, stats, gamma, beta):
    return pl.pallas_call(
        _fin_body,
        out_shape=jax.ShapeDtypeStruct((B * M, CO), jnp.float32),
    )(maxo, mino, stats, gamma, beta)


def kernel(loc, new_loc, features, W1, gamma1, beta1):
    locT = jnp.transpose(loc, (0, 2, 1))
    idx = _topk_call(locT, new_loc)
    featT = jnp.transpose(features, (0, 2, 1))
    table = jnp.concatenate(
        [loc.reshape(B * N, D), featT.reshape(B * N, C),
         jnp.zeros((B * N, CPAD - CIN), jnp.float32)], axis=1)
    g = _gather_call(idx.reshape(-1), table)
    w1p = jnp.zeros((CPAD, CO), jnp.float32).at[:CIN, :].set(W1.T)
    maxo, mino, stats = _mlp_call(g, w1p, new_loc.reshape(B * M, D))
    out = _fin_call(maxo, mino, stats, gamma1.reshape(1, CO),
                    beta1.reshape(1, CO))
    return jnp.transpose(out.reshape(B, M, CO), (0, 2, 1))


# speedup vs baseline: 12.1259x; 1.4002x over previous
import functools

import jax
import jax.numpy as jnp
from jax import lax
from jax.experimental import pallas as pl
from jax.experimental.pallas import tpu as pltpu
from jax.experimental.pallas import tpu_sc as plsc

B, N, M, K, D, C = 4, 8192, 1024, 32, 3, 32
CIN = D + C
CPAD = 48
CO = 64
QT = 128
RT = QT * K

SC_CORES = 2
SC_SUBCORES = 16
NW = SC_CORES * SC_SUBCORES
ROWS = B * M * K
ROWS_PER_W = ROWS // NW
GCHUNK = 128
NCHUNK = ROWS_PER_W // GCHUNK


SEG = 64
LN = 128
R = 5
INF = float("inf")
BIG = 2**30


def _topk_body(locT_ref, q_ref, idx_ref, d_ref, cv_ref, ci_ref):
    b = pl.program_id(0)
    p = locT_ref[0]
    px, py, pz = p[0], p[1], p[2]
    q = q_ref[0]
    qx = q[:, 0:1].reshape(QT, 1, 1)
    qy = q[:, 1:2].reshape(QT, 1, 1)
    qz = q[:, 2:3].reshape(QT, 1, 1)
    p2 = (px * px + py * py + pz * pz)[None]
    q2 = qx * qx + qy * qy + qz * qz
    bf = lambda x: x.astype(jnp.bfloat16).astype(jnp.float32)
    cross = bf(qx) * bf(px)[None] + bf(qy) * bf(py)[None] + bf(qz) * bf(pz)[None]
    d2 = jnp.maximum(q2 + p2 - 2.0 * cross, 0.0)

    s_iota = lax.broadcasted_iota(jnp.int32, (1, SEG, LN), 1)
    n_iota = s_iota * LN + lax.broadcasted_iota(jnp.int32, (1, SEG, LN), 2)
    l_iota = lax.broadcasted_iota(jnp.int32, (1, LN), 1)
    iota_k = lax.broadcasted_iota(jnp.int32, (1, K), 1)

    def lane_min_arg(d):
        s = jnp.min(d, axis=1)
        a = jnp.min(jnp.where(d == s[:, None, :], s_iota, BIG), axis=1)
        return s, a

    d_ref[...] = d2
    s_min, a_min = lane_min_arg(d2)
    for r in range(R):
        cv_ref[:, r, :] = s_min
        ci_ref[:, r, :] = a_min * LN + l_iota
        d = d_ref[...]
        kn = jnp.where(s_iota == a_min[:, None, :], INF, d)
        d_ref[...] = kn
        s_min, a_min = lane_min_arg(kn)

    m_rem = jnp.min(s_min, axis=1)
    cnt = jnp.sum((cv_ref[...] < m_rem[:, None, None]).astype(jnp.int32),
                  axis=(1, 2))
    ok = jnp.all(cnt >= K)

    def fast(_):
        def body(i, carry):
            v, acc = carry
            m = jnp.min(jnp.min(v, axis=1), axis=1)[:, None, None]
            n = jnp.min(jnp.min(jnp.where(v == m, ci_ref[...], BIG), axis=1),
                        axis=1)
            v = jnp.where(ci_ref[...] == n[:, None, None], INF, v)
            acc = jnp.where(iota_k == i, n[:, None], acc)
            return v, acc
        _, acc = lax.fori_loop(0, K, body,
                               (cv_ref[...], jnp.zeros((QT, K), jnp.int32)))
        return acc

    def slow(_):
        d_ref[...] = d2

        def body(i, acc):
            d = d_ref[...]
            m = jnp.min(jnp.min(d, axis=1), axis=1)[:, None, None]
            n = jnp.min(jnp.min(jnp.where(d == m, n_iota, BIG), axis=1),
                        axis=1)
            d_ref[...] = jnp.where(n_iota == n[:, None, None], INF, d)
            return jnp.where(iota_k == i, n[:, None], acc)

        return lax.fori_loop(0, K, body, jnp.zeros((QT, K), jnp.int32))

    acc = lax.cond(ok, fast, slow, 0)
    idx_ref[0] = acc + b * N


def _topk_call(locT, new_loc):
    return pl.pallas_call(
        _topk_body,
        grid=(B, M // QT),
        in_specs=[
            pl.BlockSpec((1, D, SEG, LN), lambda b, j: (b, 0, 0, 0)),
            pl.BlockSpec((1, QT, D), lambda b, j: (b, j, 0)),
        ],
        out_specs=pl.BlockSpec((1, QT, K), lambda b, j: (b, j, 0)),
        out_shape=jax.ShapeDtypeStruct((B, M, K), jnp.int32),
        scratch_shapes=[
            pltpu.VMEM((QT, SEG, LN), jnp.float32),
            pltpu.VMEM((QT, R, LN), jnp.float32),
            pltpu.VMEM((QT, R, LN), jnp.int32),
        ],
    )(locT, new_loc)


def _gather_kernel(idx_hbm, table_hbm, out_hbm, idx_v, rows_v, sem):
    wid = lax.axis_index("s") * SC_CORES + lax.axis_index("c")
    base = wid * ROWS_PER_W
    pltpu.sync_copy(idx_hbm.at[wid], idx_v)
    for cidx in range(NCHUNK):
        pltpu.async_copy(table_hbm.at[idx_v.at[cidx]], rows_v, sem).wait()
        pltpu.sync_copy(rows_v, out_hbm.at[pl.ds(base + cidx * GCHUNK, GCHUNK)])


def _gather_call(idx_flat, table):
    mesh = plsc.VectorSubcoreMesh(core_axis_name="c", subcore_axis_name="s")
    f = pl.kernel(
        _gather_kernel,
        out_type=jax.ShapeDtypeStruct((ROWS, CPAD), jnp.float32),
        mesh=mesh,
        compiler_params=pltpu.CompilerParams(use_tc_tiling_on_sc=False),
        scratch_types=[
            pltpu.VMEM((NCHUNK, GCHUNK), jnp.int32),
            pltpu.VMEM((GCHUNK, CPAD), jnp.float32),
            pltpu.SemaphoreType.DMA,
        ],
    )
    return f(idx_flat.reshape(NW, NCHUNK, GCHUNK), table)


def _mlp_body(g_ref, w_ref, q_ref, maxo_ref, mino_ref, stats_ref):
    t = pl.program_id(0)
    w = w_ref[...].astype(jnp.bfloat16)
    qpad = jnp.concatenate(
        [q_ref[...], jnp.zeros((QT, CPAD - D), jnp.float32)], axis=1)
    h = g_ref[...].reshape(QT, K, CPAD) - qpad[:, None, :]
    hb = h.reshape(RT, CPAD).astype(jnp.bfloat16)
    y = jnp.dot(hb, w, preferred_element_type=jnp.float32)
    y3 = y.reshape(QT, K, CO)
    maxo_ref[...] = jnp.max(y3, axis=1)
    mino_ref[...] = jnp.min(y3, axis=1)
    s = jnp.sum(y3, axis=(0, 1)).reshape(1, CO)
    s2 = jnp.sum(y3 * y3, axis=(0, 1)).reshape(1, CO)

    @pl.when(t == 0)
    def _():
        stats_ref[...] = jnp.zeros((8, CO), jnp.float32)

    stats_ref[0:1, :] += s
    stats_ref[1:2, :] += s2


def _mlp_call(g, w1p, q_flat):
    return pl.pallas_call(
        _mlp_body,
        grid=(ROWS // RT,),
        in_specs=[
            pl.BlockSpec((RT, CPAD), lambda t: (t, 0)),
            pl.BlockSpec((CPAD, CO), lambda t: (0, 0)),
            pl.BlockSpec((QT, D), lambda t: (t, 0)),
        ],
        out_specs=[
            pl.BlockSpec((QT, CO), lambda t: (t, 0)),
            pl.BlockSpec((QT, CO), lambda t: (t, 0)),
            pl.BlockSpec((8, CO), lambda t: (0, 0)),
        ],
        out_shape=[
            jax.ShapeDtypeStruct((B * M, CO), jnp.float32),
            jax.ShapeDtypeStruct((B * M, CO), jnp.float32),
            jax.ShapeDtypeStruct((8, CO), jnp.float32),
        ],
    )(g, w1p, q_flat)


def _fin_body(maxo_ref, mino_ref, stats_ref, g_ref, b_ref, out_ref):
    cnt = jnp.float32(ROWS)
    mean = stats_ref[0:1, :] / cnt
    var = stats_ref[1:2, :] / cnt - mean * mean
    scale = g_ref[...] * lax.rsqrt(var + 1e-5)
    y = jnp.where(scale > 0.0, maxo_ref[...], mino_ref[...])
    out_ref[...] = jnp.maximum((y - mean) * scale + b_ref[...], 0.0)


def _fin_call(maxo, mino, stats, gamma, beta):
    return pl.pallas_call(
        _fin_body,
        out_shape=jax.ShapeDtypeStruct((B * M, CO), jnp.float32),
    )(maxo, mino, stats, gamma, beta)


def kernel(loc, new_loc, features, W1, gamma1, beta1):
    locT = jnp.transpose(loc, (0, 2, 1)).reshape(B, D, SEG, LN)
    idx = _topk_call(locT, new_loc)
    featT = jnp.transpose(features, (0, 2, 1))
    table = jnp.concatenate(
        [loc.reshape(B * N, D), featT.reshape(B * N, C),
         jnp.zeros((B * N, CPAD - CIN), jnp.float32)], axis=1)
    g = _gather_call(idx.reshape(-1), table)
    w1p = jnp.zeros((CPAD, CO), jnp.float32).at[:CIN, :].set(W1.T)
    maxo, mino, stats = _mlp_call(g, w1p, new_loc.reshape(B * M, D))
    out = _fin_call(maxo, mino, stats, gamma1.reshape(1, CO),
                    beta1.reshape(1, CO))
    return jnp.transpose(out.reshape(B, M, CO), (0, 2, 1))
